# baseline (speedup 1.0000x reference)
import functools

import jax
import jax.numpy as jnp
from jax import lax
from jax.experimental import pallas as pl
from jax.experimental.pallas import tpu as pltpu
from jax.experimental.pallas import tpu_sc as plsc

_N_EXPERTS = 64
_TOP_K = 2
_D_IN = 768
_D_OUT = 768
_B = 4
_S = 2048
_LOSS_COEF = 0.01

_POOL_BLK = 512
_MM_BLK = 512
_L = 16


def _pool_logits_body(x_ref, wg_ref, logits_ref, acc_ref):
    i = pl.program_id(0)
    n = pl.num_programs(0)

    @pl.when(i == 0)
    def _init():
        acc_ref[...] = jnp.zeros_like(acc_ref)

    acc_ref[...] += jnp.sum(x_ref[...], axis=1)

    @pl.when(i == n - 1)
    def _finish():
        pooled = acc_ref[...] * (1.0 / _S)
        logits_ref[...] = jax.lax.dot_general(
            pooled, wg_ref[...], (((1,), (0,)), ((), ())),
            preferred_element_type=jnp.float32)


def _lane_reduce(v, op):
    r = v[0]
    for l in range(1, _L):
        r = op(r, v[l])
    return r


def _sc_routing(logits_hbm, bias_hbm, idx_out, gv_out, bmix_out, loss_out,
                logits_v, idx8_v, gv8_v, rows_v, bmix_v, loss_v, sem):
    wid = lax.axis_index("s") * 2 + lax.axis_index("c")

    @pl.when(wid == 0)
    def _route():
        pltpu.sync_copy(logits_hbm, logits_v)
        lane = lax.iota(jnp.int32, 16)
        neg = jnp.float32(-3.0e38)
        nj = _N_EXPERTS // _L
        imp = [jnp.zeros((_L,), jnp.float32) for _ in range(nj)]
        lod = [jnp.zeros((_L,), jnp.float32) for _ in range(nj)]
        gs = []
        idxvec = jnp.zeros((_L,), jnp.int32)
        gvvec = jnp.zeros((_L,), jnp.float32)
        for b in range(_B):
            vs = [logits_v[b, pl.ds(_L * j, _L)] for j in range(nj)]
            ids = [lane + _L * j for j in range(nj)]
            vmax = vs[0]
            for j in range(1, nj):
                vmax = jnp.maximum(vmax, vs[j])
            m1 = _lane_reduce(vmax, jnp.maximum)
            cand = jnp.where(vs[0] == m1, ids[0], _N_EXPERTS)
            for j in range(1, nj):
                cand = jnp.minimum(cand,
                                   jnp.where(vs[j] == m1, ids[j], _N_EXPERTS))
            a1 = _lane_reduce(cand, jnp.minimum)
            vs2 = [jnp.where(ids[j] == a1, neg, vs[j]) for j in range(nj)]
            vmax2 = vs2[0]
            for j in range(1, nj):
                vmax2 = jnp.maximum(vmax2, vs2[j])
            m2 = _lane_reduce(vmax2, jnp.maximum)
            cand2 = jnp.where(vs2[0] == m2, ids[0], _N_EXPERTS)
            for j in range(1, nj):
                cand2 = jnp.minimum(cand2,
                                    jnp.where(vs2[j] == m2, ids[j], _N_EXPERTS))
            a2 = _lane_reduce(cand2, jnp.minimum)
            ev = jnp.exp(jnp.where(lane == 0, m2 - m1, 0.0))
            inv = jnp.where(lane == 0, 1.0, 0.0) / (1.0 + ev)
            g1 = inv[0]
            g2 = 1.0 - g1
            idxvec = jnp.where(lane == 2 * b, a1, idxvec)
            idxvec = jnp.where(lane == 2 * b + 1, a2, idxvec)
            gvvec = jnp.where(lane == 2 * b, g1, gvvec)
            gvvec = jnp.where(lane == 2 * b + 1, g2, gvvec)
            gs += [g1, g2]
            for j in range(nj):
                imp[j] = (imp[j] + jnp.where(ids[j] == a1, g1, 0.0)
                          + jnp.where(ids[j] == a2, g2, 0.0))
                lod[j] = (lod[j] + jnp.where(ids[j] == a1, 1.0, 0.0)
                          + jnp.where(ids[j] == a2, 1.0, 0.0))

        def var_and_den(vecs):
            tot = vecs[0]
            for j in range(1, nj):
                tot = tot + vecs[j]
            mu = _lane_reduce(tot, jnp.add) * (1.0 / _N_EXPERTS)
            sq = jnp.zeros((_L,), jnp.float32)
            for j in range(nj):
                dj = vecs[j] - mu
                sq = sq + dj * dj
            var = _lane_reduce(sq, jnp.add) * (1.0 / (_N_EXPERTS - 1))
            return var, mu * mu + 1e-10

        var_i, den_i = var_and_den(imp)
        var_l, den_l = var_and_den(lod)
        numv = jnp.where(lane == 0, var_i, jnp.where(lane == 1, var_l, 0.0))
        denv = jnp.where(lane == 0, den_i, jnp.where(lane == 1, den_l, 1.0))
        q = numv / denv
        loss = (q[0] + q[1]) * _LOSS_COEF
        loss_v[...] = jnp.where(lane == 0, loss, 0.0)
        idx8_v[...] = idxvec
        gv8_v[...] = gvvec

        pltpu.async_copy(bias_hbm.at[idx8_v], rows_v, sem).wait()
        for b in range(_B):
            for j in range(_D_OUT // _L):
                sl = pl.ds(_L * j, _L)
                bmix_v[b, 0, sl] = (gs[2 * b] * rows_v[2 * b, sl]
                                    + gs[2 * b + 1] * rows_v[2 * b + 1, sl])
        pltpu.sync_copy(idx8_v, idx_out)
        pltpu.sync_copy(gv8_v, gv_out)
        pltpu.sync_copy(bmix_v, bmix_out)
        pltpu.sync_copy(loss_v, loss_out)


def _mix_matmul_body(idx_ref, x_ref, w0_ref, w1_ref, bm_ref, g_ref,
                     o_ref, wmix_ref):
    b = pl.program_id(0)
    s = pl.program_id(1)
    g0 = g_ref[2 * b]
    g1 = g_ref[2 * b + 1]

    @pl.when(s == 0)
    def _mix():
        wmix_ref[...] = (g0 * w0_ref[0] + g1 * w1_ref[0]).astype(jnp.bfloat16)

    y = jax.lax.dot_general(
        x_ref[0].astype(jnp.bfloat16), wmix_ref[...],
        (((1,), (1,)), ((), ())),
        preferred_element_type=jnp.float32)
    o_ref[0] = y + bm_ref[0]


def kernel(x, w_gate, weight, bias):
    n_pool = _S // _POOL_BLK
    logits = pl.pallas_call(
        _pool_logits_body,
        grid=(n_pool,),
        in_specs=[
            pl.BlockSpec((_B, _POOL_BLK, _D_IN), lambda i: (0, i, 0)),
            pl.BlockSpec((_D_IN, _N_EXPERTS), lambda i: (0, 0)),
        ],
        out_specs=pl.BlockSpec((_B, _N_EXPERTS), lambda i: (0, 0)),
        out_shape=jax.ShapeDtypeStruct((_B, _N_EXPERTS), jnp.float32),
        scratch_shapes=[pltpu.VMEM((_B, _D_IN), jnp.float32)],
    )(x, w_gate)

    mesh = plsc.VectorSubcoreMesh(core_axis_name="c", subcore_axis_name="s")
    route = functools.partial(
        pl.kernel,
        mesh=mesh,
        out_type=[
            jax.ShapeDtypeStruct((_L,), jnp.int32),
            jax.ShapeDtypeStruct((_L,), jnp.float32),
            jax.ShapeDtypeStruct((_B, 1, _D_OUT), jnp.float32),
            jax.ShapeDtypeStruct((_L,), jnp.float32),
        ],
        scratch_types=[
            pltpu.VMEM((_B, _N_EXPERTS), jnp.float32),
            pltpu.VMEM((_L,), jnp.int32),
            pltpu.VMEM((_L,), jnp.float32),
            pltpu.VMEM((_L, _D_OUT), jnp.float32),
            pltpu.VMEM((_B, 1, _D_OUT), jnp.float32),
            pltpu.VMEM((_L,), jnp.float32),
            pltpu.SemaphoreType.DMA,
        ],
    )(_sc_routing)
    idx8, gv8, bias_mix, loss_v = route(logits, bias)

    n_mm = _S // _MM_BLK
    grid_spec = pltpu.PrefetchScalarGridSpec(
        num_scalar_prefetch=1,
        grid=(_B, n_mm),
        in_specs=[
            pl.BlockSpec((1, _MM_BLK, _D_IN), lambda b, s, idx: (b, s, 0)),
            pl.BlockSpec((1, _D_OUT, _D_IN),
                         lambda b, s, idx: (idx[2 * b], 0, 0)),
            pl.BlockSpec((1, _D_OUT, _D_IN),
                         lambda b, s, idx: (idx[2 * b + 1], 0, 0)),
            pl.BlockSpec((1, 1, _D_OUT), lambda b, s, idx: (b, 0, 0)),
            pl.BlockSpec(memory_space=pltpu.SMEM),
        ],
        out_specs=pl.BlockSpec((1, _MM_BLK, _D_OUT), lambda b, s, idx: (b, s, 0)),
        scratch_shapes=[pltpu.VMEM((_D_OUT, _D_IN), jnp.bfloat16)],
    )
    y = pl.pallas_call(
        _mix_matmul_body,
        grid_spec=grid_spec,
        out_shape=jax.ShapeDtypeStruct((_B, _S, _D_OUT), jnp.float32),
    )(idx8, x, weight, weight, bias_mix, gv8)

    return (y, loss_v[0])


# speedup vs baseline: 1.5923x; 1.5923x over previous
import jax
import jax.numpy as jnp
from jax.experimental import pallas as pl
from jax.experimental.pallas import tpu as pltpu

_N_EXPERTS = 64
_TOP_K = 2
_D_IN = 768
_D_OUT = 768
_B = 4
_S = 2048
_LOSS_COEF = 0.01

_BLK = 256
_NCH = _S // _BLK


def _fused_body(x_ref, wg_ref, b_ref, w_hbm, o_ref, loss_ref,
                xb_ref, acc_ref, wbuf_ref, wmix_ref, bmix_ref,
                idx_s, g_s, sems):
    i = pl.program_id(0)

    @pl.when(i < _NCH)
    def _pool():
        @pl.when(i == 0)
        def _init():
            acc_ref[...] = jnp.zeros_like(acc_ref)

        acc_ref[...] += jnp.sum(x_ref[...], axis=1)
        xb_ref[i] = x_ref[...].astype(jnp.bfloat16)

    @pl.when(i == _NCH - 1)
    def _gate():
        pooled = acc_ref[...] * (1.0 / _S)
        logits = jax.lax.dot_general(
            pooled, wg_ref[...], (((1,), (0,)), ((), ())),
            preferred_element_type=jnp.float32)
        iota = jax.lax.broadcasted_iota(jnp.int32, (_B, _N_EXPERTS), 1)
        m1 = jnp.max(logits, axis=1, keepdims=True)
        a1 = jnp.min(jnp.where(logits == m1, iota, _N_EXPERTS), axis=1,
                     keepdims=True)
        l2 = jnp.where(iota == a1, -jnp.inf, logits)
        m2 = jnp.max(l2, axis=1, keepdims=True)
        a2 = jnp.min(jnp.where(l2 == m2, iota, _N_EXPERTS), axis=1,
                     keepdims=True)
        e = jnp.exp(m2 - m1)
        g1 = 1.0 / (1.0 + e)
        g2 = e / (1.0 + e)
        gates = (jnp.where(iota == a1, g1, 0.0)
                 + jnp.where(iota == a2, g2, 0.0))
        importance = jnp.sum(gates, axis=0, keepdims=True)
        load = jnp.sum((gates > 0).astype(jnp.float32), axis=0, keepdims=True)

        def cv_sq(v):
            mu = jnp.mean(v)
            var = jnp.sum((v - mu) ** 2) * (1.0 / (_N_EXPERTS - 1))
            return var / (mu * mu + 1e-10)

        loss_ref[0] = (cv_sq(importance) + cv_sq(load)) * _LOSS_COEF
        bmix = jax.lax.dot_general(
            gates, b_ref[...], (((1,), (0,)), ((), ())),
            preferred_element_type=jnp.float32)
        bmix_ref[...] = bmix[:, None, :]

        for bb in range(_B):
            i1 = jnp.max(jnp.where(a1[bb:bb + 1, :] < _N_EXPERTS,
                                   a1[bb:bb + 1, :], 0))
            i2 = jnp.max(jnp.where(a2[bb:bb + 1, :] < _N_EXPERTS,
                                   a2[bb:bb + 1, :], 0))
            idx_s[2 * bb] = i1
            idx_s[2 * bb + 1] = i2
            g_s[2 * bb] = jnp.max(g1[bb:bb + 1, :])
            g_s[2 * bb + 1] = jnp.max(g2[bb:bb + 1, :])
            pltpu.make_async_copy(
                w_hbm.at[i1], wbuf_ref.at[bb, 0],
                sems.at[bb, 0]).start()
            pltpu.make_async_copy(
                w_hbm.at[i2], wbuf_ref.at[bb, 1],
                sems.at[bb, 1]).start()

    @pl.when(i >= _NCH)
    def _matmul():
        j = i - _NCH
        b = j // _NCH
        s = j % _NCH

        @pl.when(s == 0)
        def _mix():
            pltpu.make_async_copy(
                w_hbm.at[0], wbuf_ref.at[b, 0], sems.at[b, 0]).wait()
            pltpu.make_async_copy(
                w_hbm.at[0], wbuf_ref.at[b, 1], sems.at[b, 1]).wait()
            g0 = g_s[2 * b]
            g1v = g_s[2 * b + 1]
            wmix_ref[...] = (g0 * wbuf_ref[b, 0]
                             + g1v * wbuf_ref[b, 1]).astype(jnp.bfloat16)

        y = jax.lax.dot_general(
            xb_ref[s, b], wmix_ref[...], (((1,), (1,)), ((), ())),
            preferred_element_type=jnp.float32)
        o_ref[0] = y + bmix_ref[b]


def kernel(x, w_gate, weight, bias):
    def _x_map(i):
        return (0, jnp.minimum(i, _NCH - 1), 0)

    def _o_map(i):
        j = jnp.maximum(i - _NCH, 0)
        return (j // _NCH, j % _NCH, 0)

    y, loss_arr = pl.pallas_call(
        _fused_body,
        grid=(_NCH + _B * _NCH,),
        in_specs=[
            pl.BlockSpec((_B, _BLK, _D_IN), _x_map),
            pl.BlockSpec((_D_IN, _N_EXPERTS), lambda i: (0, 0)),
            pl.BlockSpec((_N_EXPERTS, _D_OUT), lambda i: (0, 0)),
            pl.BlockSpec(memory_space=pl.ANY),
        ],
        out_specs=[
            pl.BlockSpec((1, _BLK, _D_OUT), _o_map),
            pl.BlockSpec(memory_space=pltpu.SMEM),
        ],
        out_shape=[
            jax.ShapeDtypeStruct((_B, _S, _D_OUT), jnp.float32),
            jax.ShapeDtypeStruct((1,), jnp.float32),
        ],
        scratch_shapes=[
            pltpu.VMEM((_NCH, _B, _BLK, _D_IN), jnp.bfloat16),
            pltpu.VMEM((_B, _D_IN), jnp.float32),
            pltpu.VMEM((_B, _TOP_K, _D_OUT, _D_IN), jnp.float32),
            pltpu.VMEM((_D_OUT, _D_IN), jnp.bfloat16),
            pltpu.VMEM((_B, 1, _D_OUT), jnp.float32),
            pltpu.SMEM((2 * _B,), jnp.int32),
            pltpu.SMEM((2 * _B,), jnp.float32),
            pltpu.SemaphoreType.DMA((_B, _TOP_K)),
        ],
    )(x, w_gate, bias, weight)

    return (y, loss_arr[0])


# speedup vs baseline: 2.1542x; 1.3529x over previous
import jax
import jax.numpy as jnp
from jax.experimental import pallas as pl
from jax.experimental.pallas import tpu as pltpu

_N_EXPERTS = 64
_TOP_K = 2
_D_IN = 768
_D_OUT = 768
_B = 4
_S = 2048
_LOSS_COEF = 0.01

_BLK = 512
_NCH = _S // _BLK
_MM_BLK = 1024
_MM_STEPS = _S // _MM_BLK


def _fused_body(x_ref, wg_ref, b_ref, w_hbm, o_ref, loss_ref,
                xb_ref, acc_ref, wbuf_ref, wmix_ref, bmix_ref,
                idx_s, g_s, sems):
    i = pl.program_id(0)

    @pl.when(i < _NCH)
    def _pool():
        @pl.when(i == 0)
        def _init():
            acc_ref[...] = jnp.zeros_like(acc_ref)

        acc_ref[...] += jnp.sum(x_ref[...], axis=1)
        xb_ref[i] = x_ref[...].astype(jnp.bfloat16)

    @pl.when(i == _NCH - 1)
    def _gate():
        pooled = acc_ref[...] * (1.0 / _S)
        logits = jax.lax.dot_general(
            pooled, wg_ref[...], (((1,), (0,)), ((), ())),
            preferred_element_type=jnp.float32)
        iota = jax.lax.broadcasted_iota(jnp.int32, (_B, _N_EXPERTS), 1)
        m1 = jnp.max(logits, axis=1, keepdims=True)
        a1 = jnp.min(jnp.where(logits == m1, iota, _N_EXPERTS), axis=1,
                     keepdims=True)
        l2 = jnp.where(iota == a1, -jnp.inf, logits)
        m2 = jnp.max(l2, axis=1, keepdims=True)
        a2 = jnp.min(jnp.where(l2 == m2, iota, _N_EXPERTS), axis=1,
                     keepdims=True)
        e = jnp.exp(m2 - m1)
        g1 = 1.0 / (1.0 + e)
        g2 = e / (1.0 + e)
        gates = (jnp.where(iota == a1, g1, 0.0)
                 + jnp.where(iota == a2, g2, 0.0))
        importance = jnp.sum(gates, axis=0, keepdims=True)
        load = jnp.sum((gates > 0).astype(jnp.float32), axis=0, keepdims=True)

        def cv_sq(v):
            mu = jnp.mean(v)
            var = jnp.sum((v - mu) ** 2) * (1.0 / (_N_EXPERTS - 1))
            return var / (mu * mu + 1e-10)

        loss_ref[0] = (cv_sq(importance) + cv_sq(load)) * _LOSS_COEF
        bmix = jax.lax.dot_general(
            gates, b_ref[...], (((1,), (0,)), ((), ())),
            preferred_element_type=jnp.float32)
        bmix_ref[...] = bmix[:, None, :]

        for bb in range(_B):
            i1 = jnp.max(jnp.where(a1[bb:bb + 1, :] < _N_EXPERTS,
                                   a1[bb:bb + 1, :], 0))
            i2 = jnp.max(jnp.where(a2[bb:bb + 1, :] < _N_EXPERTS,
                                   a2[bb:bb + 1, :], 0))
            idx_s[2 * bb] = i1
            idx_s[2 * bb + 1] = i2
            g_s[2 * bb] = jnp.max(g1[bb:bb + 1, :])
            g_s[2 * bb + 1] = jnp.max(g2[bb:bb + 1, :])
            pltpu.make_async_copy(
                w_hbm.at[i1], wbuf_ref.at[bb, 0],
                sems.at[bb, 0]).start()
            pltpu.make_async_copy(
                w_hbm.at[i2], wbuf_ref.at[bb, 1],
                sems.at[bb, 1]).start()

    @pl.when(i >= _NCH)
    def _matmul():
        j = i - _NCH
        b = j // _MM_STEPS
        s = j % _MM_STEPS

        @pl.when(s == 0)
        def _mix():
            pltpu.make_async_copy(
                w_hbm.at[0], wbuf_ref.at[b, 0], sems.at[b, 0]).wait()
            pltpu.make_async_copy(
                w_hbm.at[0], wbuf_ref.at[b, 1], sems.at[b, 1]).wait()
            g0 = g_s[2 * b]
            g1v = g_s[2 * b + 1]
            wmix_ref[...] = (g0 * wbuf_ref[b, 0]
                             + g1v * wbuf_ref[b, 1]).astype(jnp.bfloat16)

        for h in range(_MM_BLK // _BLK):
            y = jax.lax.dot_general(
                xb_ref[s * (_MM_BLK // _BLK) + h, b], wmix_ref[...],
                (((1,), (1,)), ((), ())),
                preferred_element_type=jnp.float32)
            o_ref[0, pl.ds(h * _BLK, _BLK)] = y + bmix_ref[b]


def kernel(x, w_gate, weight, bias):
    def _x_map(i):
        return (0, jnp.minimum(i, _NCH - 1), 0)

    def _o_map(i):
        j = jnp.maximum(i - _NCH, 0)
        return (j // _MM_STEPS, j % _MM_STEPS, 0)

    y, loss_arr = pl.pallas_call(
        _fused_body,
        grid=(_NCH + _B * _MM_STEPS,),
        in_specs=[
            pl.BlockSpec((_B, _BLK, _D_IN), _x_map),
            pl.BlockSpec((_D_IN, _N_EXPERTS), lambda i: (0, 0)),
            pl.BlockSpec((_N_EXPERTS, _D_OUT), lambda i: (0, 0)),
            pl.BlockSpec(memory_space=pl.ANY),
        ],
        out_specs=[
            pl.BlockSpec((1, _MM_BLK, _D_OUT), _o_map),
            pl.BlockSpec(memory_space=pltpu.SMEM),
        ],
        out_shape=[
            jax.ShapeDtypeStruct((_B, _S, _D_OUT), jnp.float32),
            jax.ShapeDtypeStruct((1,), jnp.float32),
        ],
        scratch_shapes=[
            pltpu.VMEM((_NCH, _B, _BLK, _D_IN), jnp.bfloat16),
            pltpu.VMEM((_B, _D_IN), jnp.float32),
            pltpu.VMEM((_B, _TOP_K, _D_OUT, _D_IN), jnp.float32),
            pltpu.VMEM((_D_OUT, _D_IN), jnp.bfloat16),
            pltpu.VMEM((_B, 1, _D_OUT), jnp.float32),
            pltpu.SMEM((2 * _B,), jnp.int32),
            pltpu.SMEM((2 * _B,), jnp.float32),
            pltpu.SemaphoreType.DMA((_B, _TOP_K)),
        ],
    )(x, w_gate, bias, weight)

    return (y, loss_arr[0])


# speedup vs baseline: 2.2748x; 1.0560x over previous
import jax
import jax.numpy as jnp
from jax.experimental import pallas as pl
from jax.experimental.pallas import tpu as pltpu

_N_EXPERTS = 64
_TOP_K = 2
_D_IN = 768
_D_OUT = 768
_B = 4
_S = 2048
_LOSS_COEF = 0.01

_BLK = 512
_NCH = _S // _BLK
_MM_BLK = 2048
_MM_STEPS = _S // _MM_BLK


def _fused_body(x_ref, wg_ref, b_ref, w_hbm, o_ref, loss_ref,
                xb_ref, acc_ref, wbuf_ref, wmix_ref, bmix_ref,
                idx_s, g_s, sems):
    i = pl.program_id(0)

    @pl.when(i < _NCH)
    def _pool():
        @pl.when(i == 0)
        def _init():
            acc_ref[...] = jnp.zeros_like(acc_ref)

        acc_ref[...] += jnp.sum(x_ref[...], axis=1)
        xb_ref[i] = x_ref[...].astype(jnp.bfloat16)

    @pl.when(i == _NCH - 1)
    def _gate():
        pooled = acc_ref[...] * (1.0 / _S)
        logits = jax.lax.dot_general(
            pooled, wg_ref[...], (((1,), (0,)), ((), ())),
            preferred_element_type=jnp.float32)
        iota = jax.lax.broadcasted_iota(jnp.int32, (_B, _N_EXPERTS), 1)
        m1 = jnp.max(logits, axis=1, keepdims=True)
        a1 = jnp.min(jnp.where(logits == m1, iota, _N_EXPERTS), axis=1,
                     keepdims=True)
        l2 = jnp.where(iota == a1, -jnp.inf, logits)
        m2 = jnp.max(l2, axis=1, keepdims=True)
        a2 = jnp.min(jnp.where(l2 == m2, iota, _N_EXPERTS), axis=1,
                     keepdims=True)
        e = jnp.exp(m2 - m1)
        g1 = 1.0 / (1.0 + e)
        g2 = e / (1.0 + e)
        gates = (jnp.where(iota == a1, g1, 0.0)
                 + jnp.where(iota == a2, g2, 0.0))
        importance = jnp.sum(gates, axis=0, keepdims=True)
        load = jnp.sum((gates > 0).astype(jnp.float32), axis=0, keepdims=True)

        def cv_sq(v):
            mu = jnp.mean(v)
            var = jnp.sum((v - mu) ** 2) * (1.0 / (_N_EXPERTS - 1))
            return var / (mu * mu + 1e-10)

        loss_ref[0] = (cv_sq(importance) + cv_sq(load)) * _LOSS_COEF
        bmix = jax.lax.dot_general(
            gates, b_ref[...], (((1,), (0,)), ((), ())),
            preferred_element_type=jnp.float32)
        bmix_ref[...] = bmix[:, None, :]

        for bb in range(_B):
            i1 = jnp.max(jnp.where(a1[bb:bb + 1, :] < _N_EXPERTS,
                                   a1[bb:bb + 1, :], 0))
            i2 = jnp.max(jnp.where(a2[bb:bb + 1, :] < _N_EXPERTS,
                                   a2[bb:bb + 1, :], 0))
            idx_s[2 * bb] = i1
            idx_s[2 * bb + 1] = i2
            g_s[2 * bb] = jnp.max(g1[bb:bb + 1, :])
            g_s[2 * bb + 1] = jnp.max(g2[bb:bb + 1, :])
            pltpu.make_async_copy(
                w_hbm.at[i1], wbuf_ref.at[bb, 0],
                sems.at[bb, 0]).start()
            pltpu.make_async_copy(
                w_hbm.at[i2], wbuf_ref.at[bb, 1],
                sems.at[bb, 1]).start()

    @pl.when(i >= _NCH)
    def _matmul():
        j = i - _NCH
        b = j // _MM_STEPS
        s = j % _MM_STEPS

        @pl.when(s == 0)
        def _mix():
            pltpu.make_async_copy(
                w_hbm.at[0], wbuf_ref.at[b, 0], sems.at[b, 0]).wait()
            pltpu.make_async_copy(
                w_hbm.at[0], wbuf_ref.at[b, 1], sems.at[b, 1]).wait()
            g0 = g_s[2 * b]
            g1v = g_s[2 * b + 1]
            wmix_ref[...] = (g0 * wbuf_ref[b, 0]
                             + g1v * wbuf_ref[b, 1]).astype(jnp.bfloat16)

        for h in range(_MM_BLK // _BLK):
            y = jax.lax.dot_general(
                xb_ref[s * (_MM_BLK // _BLK) + h, b], wmix_ref[...],
                (((1,), (1,)), ((), ())),
                preferred_element_type=jnp.float32)
            o_ref[0, pl.ds(h * _BLK, _BLK)] = y + bmix_ref[b]


def kernel(x, w_gate, weight, bias):
    def _x_map(i):
        return (0, jnp.minimum(i, _NCH - 1), 0)

    def _o_map(i):
        j = jnp.maximum(i - _NCH, 0)
        return (j // _MM_STEPS, j % _MM_STEPS, 0)

    y, loss_arr = pl.pallas_call(
        _fused_body,
        grid=(_NCH + _B * _MM_STEPS,),
        in_specs=[
            pl.BlockSpec((_B, _BLK, _D_IN), _x_map),
            pl.BlockSpec((_D_IN, _N_EXPERTS), lambda i: (0, 0)),
            pl.BlockSpec((_N_EXPERTS, _D_OUT), lambda i: (0, 0)),
            pl.BlockSpec(memory_space=pl.ANY),
        ],
        out_specs=[
            pl.BlockSpec((1, _MM_BLK, _D_OUT), _o_map),
            pl.BlockSpec(memory_space=pltpu.SMEM),
        ],
        out_shape=[
            jax.ShapeDtypeStruct((_B, _S, _D_OUT), jnp.float32),
            jax.ShapeDtypeStruct((1,), jnp.float32),
        ],
        scratch_shapes=[
            pltpu.VMEM((_NCH, _B, _BLK, _D_IN), jnp.bfloat16),
            pltpu.VMEM((_B, _D_IN), jnp.float32),
            pltpu.VMEM((_B, _TOP_K, _D_OUT, _D_IN), jnp.float32),
            pltpu.VMEM((_D_OUT, _D_IN), jnp.bfloat16),
            pltpu.VMEM((_B, 1, _D_OUT), jnp.float32),
            pltpu.SMEM((2 * _B,), jnp.int32),
            pltpu.SMEM((2 * _B,), jnp.float32),
            pltpu.SemaphoreType.DMA((_B, _TOP_K)),
        ],
    )(x, w_gate, bias, weight)

    return (y, loss_arr[0])
